# baseline (speedup 1.0000x reference)
import functools
import math

import jax
import jax.numpy as jnp
from jax import lax
from jax.experimental import pallas as pl
from jax.experimental.pallas import tpu as pltpu
from jax.experimental.pallas import tpu_sc as plsc

D = 256
DE = 16
K = 20
H = 2
DH = D // H
TBLW = 64

_COS_C1 = 6.28125
_COS_C2 = 0.0019340515136718750
_COS_C3 = 1.2554227678489685e-06
_INV_2PI = 0.15915494309189535
_COS_POLY = (0.9999994, -0.49999544, 0.041660894, -0.001386227,
             2.424664e-05, -2.2163067e-07)


def _fast_cos(t):
    f = jnp.float32
    n = jnp.floor(t * f(_INV_2PI) + f(0.5))
    r = ((t - n * f(_COS_C1)) - n * f(_COS_C2)) - n * f(_COS_C3)
    z = r * r
    acc = jnp.full_like(z, f(_COS_POLY[-1]))
    for c in _COS_POLY[-2::-1]:
        acc = acc * z + f(c)
    return acc


def _sc_dims():
    try:
        info = plsc.get_sparse_core_info()
        return int(info.num_cores), int(info.num_subcores)
    except Exception:
        return 2, 16


def _sc_gather_tables(nodes, tbl):
    B3 = nodes.shape[0]
    NC, NS = _sc_dims()
    NW = NC * NS
    assert B3 % NW == 0
    per = B3 // NW
    mesh = plsc.VectorSubcoreMesh(
        core_axis_name="c", subcore_axis_name="s", num_cores=NC, num_subcores=NS)

    @functools.partial(
        pl.kernel,
        out_type=jax.ShapeDtypeStruct((B3, TBLW), jnp.int32),
        mesh=mesh,
        compiler_params=pltpu.CompilerParams(use_tc_tiling_on_sc=False),
        scratch_types=[
            pltpu.VMEM((per,), jnp.int32),
            pltpu.VMEM((per, TBLW), jnp.int32),
            pltpu.SemaphoreType.DMA,
        ],
    )
    def body(nodes_hbm, tbl_hbm, g_out, idx_v, tbl_v, s1):
        wid = lax.axis_index("s") * NC + lax.axis_index("c")
        base = wid * per
        pltpu.sync_copy(nodes_hbm.at[pl.ds(base, per)], idx_v)
        pltpu.async_copy(tbl_hbm.at[idx_v], tbl_v, s1).wait()
        pltpu.sync_copy(tbl_v, g_out.at[pl.ds(base, per)])

    return body(nodes, tbl)


def _sc_gather_embeddings(nodes, nidx3d, node_emb, emb_bf3, B3, M):
    NC, NS = _sc_dims()
    NW = NC * NS
    CH = 128
    assert B3 % NW == 0 and M % (NW * CH) == 0
    per_q = B3 // NW
    n_ch = M // (NW * CH)
    per = n_ch * CH
    mesh = plsc.VectorSubcoreMesh(
        core_axis_name="c", subcore_axis_name="s", num_cores=NC, num_subcores=NS)

    @functools.partial(
        pl.kernel,
        out_type=[
            jax.ShapeDtypeStruct((B3, D), jnp.float32),
            jax.ShapeDtypeStruct((M, 128), jnp.int32),
        ],
        mesh=mesh,
        scratch_types=[
            pltpu.VMEM((per_q,), jnp.int32),
            pltpu.VMEM((n_ch, CH), jnp.int32),
            pltpu.VMEM((per_q, D), jnp.float32),
            pltpu.VMEM((3, CH, 128), jnp.int32),
            pltpu.SemaphoreType.DMA,
            pltpu.SemaphoreType.DMA,
            pltpu.SemaphoreType.DMA,
            pltpu.SemaphoreType.DMA,
            pltpu.SemaphoreType.DMA,
            pltpu.SemaphoreType.DMA,
            pltpu.SemaphoreType.DMA,
        ],
    )
    def body(nodes_hbm, ni_hbm, emb_hbm, embbf_hbm, x_out, nf_out,
             qidx_v, ni_v, x_v, nbuf, sq, sg0, sg1, sg2, sw0, sw1, sw2):
        wid = lax.axis_index("s") * NC + lax.axis_index("c")
        qbase = wid * per_q
        base = wid * per
        pltpu.sync_copy(nodes_hbm.at[pl.ds(qbase, per_q)], qidx_v)
        pltpu.sync_copy(ni_hbm.at[wid], ni_v)
        cq = pltpu.async_copy(emb_hbm.at[qidx_v], x_v, sq)
        sg = (sg0, sg1, sg2)
        sw = (sw0, sw1, sw2)
        NB = 3
        gath = [None] * n_ch
        wb = [None] * n_ch
        for j in range(n_ch):
            b = j % NB
            if j >= NB:
                wb[j - NB].wait()
            gath[j] = pltpu.async_copy(embbf_hbm.at[ni_v.at[j]], nbuf.at[b], sg[b])
            if j >= 2:
                p = j - 2
                pb = p % NB
                gath[p].wait()
                wb[p] = pltpu.async_copy(
                    nbuf.at[pb], nf_out.at[pl.ds(base + p * CH, CH)], sw[pb])
        for p in (n_ch - 2, n_ch - 1):
            if p >= 0 and wb[p] is None:
                pb = p % NB
                gath[p].wait()
                wb[p] = pltpu.async_copy(
                    nbuf.at[pb], nf_out.at[pl.ds(base + p * CH, CH)], sw[pb])
        cq.wait()
        pltpu.sync_copy(x_v, x_out.at[pl.ds(qbase, per_q)])
        for p in range(max(0, n_ch - NB), n_ch):
            if wb[p] is not None:
                wb[p].wait()

    return body(nodes, nidx3d, node_emb, emb_bf3)


def _sc_gather_edges(eidx2d, edge_feat, M):
    NC, NS = _sc_dims()
    NW = NC * NS
    CH = 128
    assert M % (NW * CH) == 0
    n_ch = M // (NW * CH)
    per = n_ch * CH
    mesh = plsc.VectorSubcoreMesh(
        core_axis_name="c", subcore_axis_name="s", num_cores=NC, num_subcores=NS)

    @functools.partial(
        pl.kernel,
        out_type=jax.ShapeDtypeStruct((M, DE), jnp.float32),
        mesh=mesh,
        compiler_params=pltpu.CompilerParams(use_tc_tiling_on_sc=False),
        scratch_types=[
            pltpu.VMEM((n_ch, CH), jnp.int32),
            pltpu.VMEM((2, CH, DE), jnp.float32),
            pltpu.SemaphoreType.DMA,
            pltpu.SemaphoreType.DMA,
        ],
    )
    def body(ei_hbm, ef_hbm, ef_out, ei_v, ebuf, se0, se1):
        wid = lax.axis_index("s") * NC + lax.axis_index("c")
        base = wid * per
        pltpu.sync_copy(ei_hbm.at[pl.ds(wid * n_ch, n_ch)], ei_v)
        se = (se0, se1)
        prev = None
        for j in range(n_ch):
            b = j % 2
            ce = pltpu.async_copy(ef_hbm.at[ei_v.at[j]], ebuf.at[b], se[b])
            if prev is not None:
                pce, pj, pb = prev
                pce.wait()
                pltpu.sync_copy(ebuf.at[pb], ef_out.at[pl.ds(base + pj * CH, CH)])
            prev = (ce, j, b)
        pce, pj, pb = prev
        pce.wait()
        pltpu.sync_copy(ebuf.at[pb], ef_out.at[pl.ds(base + pj * CH, CH)])

    return body(eidx2d, edge_feat)


def _tc_body(nn_ref, rs_ref, ts_ref, nts_ref, x_ref, nf_ref, ef_ref,
             tw_ref, tb_ref, wqx_ref, wqt_ref, wkn_ref, wke_ref, wkt_ref,
             wvn_ref, wve_ref, wvt_ref, w1o_ref, w1x_ref, b1_ref, w2_ref,
             b2_ref, out_ref):
    R = ts_ref.shape[0]
    f32 = jnp.float32
    dot = functools.partial(jnp.dot, preferred_element_type=f32)
    x = x_ref[...]
    nf = nf_ref[...]
    ef = ef_ref[...]
    tw = tw_ref[...]
    tb = tb_ref[...]
    delta = ts_ref[...] - nts_ref[...]
    te = _fast_cos(delta[:, :, None] * tw.reshape(1, 1, D) + tb.reshape(1, 1, D))
    te = te.reshape(R * K, D)
    kk = dot(nf, wkn_ref[...]) + dot(ef, wke_ref[...]) + dot(te, wkt_ref[...])
    vv = dot(nf, wvn_ref[...]) + dot(ef, wve_ref[...]) + dot(te, wvt_ref[...])
    q = dot(x, wqx_ref[...]) + dot(jnp.cos(tb), wqt_ref[...])
    k3 = kk.reshape(R, K, D)
    v3 = vv.reshape(R, K, D)
    nn = nn_ref[0, 0]
    kiota = lax.broadcasted_iota(jnp.int32, (R, K), 1)
    inv_sqrt = f32(1.0 / math.sqrt(DH))
    outs = []
    for h in range(H):
        sl = slice(h * DH, (h + 1) * DH)
        q_h = q[:, sl]
        k_h = k3[:, :, sl]
        v_h = v3[:, :, sl]
        scores = jnp.sum(q_h[:, None, :] * k_h, axis=-1) * inv_sqrt
        scores = jnp.where(kiota < nn, scores, -jnp.inf)
        m = jnp.max(scores, axis=1, keepdims=True)
        e = jnp.exp(scores - m)
        attn = e / jnp.sum(e, axis=1, keepdims=True)
        outs.append(jnp.sum(attn[:, :, None] * v_h, axis=1))
    out = jnp.concatenate(outs, axis=-1)
    hh = dot(out, w1o_ref[...]) + dot(x, w1x_ref[...]) + b1_ref[...]
    hh = jnp.maximum(hh, 0.0)
    out_ref[...] = dot(hh, w2_ref[...]) + b2_ref[...] + rs_ref[0, 0] * x


def _tc_embed(R, B3, interpret=False):
    G = B3 // R
    row = lambda i: (i, 0)
    fix = lambda i: (0, 0)
    smem = pl.BlockSpec(memory_space=pltpu.SMEM)
    return pl.pallas_call(
        _tc_body,
        grid=(G,),
        in_specs=[
            smem,
            smem,
            pl.BlockSpec((R, 1), row),
            pl.BlockSpec((R, K), row),
            pl.BlockSpec((R, D), row),
            pl.BlockSpec((R * K, D), row),
            pl.BlockSpec((R * K, DE), row),
            pl.BlockSpec((1, D), fix),
            pl.BlockSpec((1, D), fix),
            pl.BlockSpec((D, D), fix),
            pl.BlockSpec((D, D), fix),
            pl.BlockSpec((D, D), fix),
            pl.BlockSpec((DE, D), fix),
            pl.BlockSpec((D, D), fix),
            pl.BlockSpec((D, D), fix),
            pl.BlockSpec((DE, D), fix),
            pl.BlockSpec((D, D), fix),
            pl.BlockSpec((D, D), fix),
            pl.BlockSpec((D, D), fix),
            pl.BlockSpec((1, D), fix),
            pl.BlockSpec((D, D), fix),
            pl.BlockSpec((1, D), fix),
        ],
        out_specs=pl.BlockSpec((R, D), row),
        out_shape=jax.ShapeDtypeStruct((B3, D), jnp.float32),
        interpret=interpret,
    )


def kernel(source_nodes, destination_nodes, negative_nodes, edge_times, edge_idxs,
           n_neighbors, node_emb, edge_feat, nbr_nodes, nbr_eidx, nbr_times,
           time_w, time_b, Wq, Wk, Wv, W1, b1, W2, b2, res_scale):
    i32 = jnp.int32
    f32 = jnp.float32
    bf16 = jnp.bfloat16
    node_emb = node_emb.astype(f32)
    edge_feat = edge_feat.astype(f32)
    ts = edge_times.astype(f32)
    emb_bf3 = lax.bitcast_convert_type(
        node_emb.astype(bf16).reshape(node_emb.shape[0], 128, 2), i32)

    Nn = nbr_nodes.shape[0]
    tbl = jnp.concatenate([
        nbr_nodes.astype(i32),
        nbr_eidx.astype(i32),
        lax.bitcast_convert_type(nbr_times.astype(f32), i32),
        jnp.zeros((Nn, TBLW - 3 * K), i32),
    ], axis=1)

    NC, NS = _sc_dims()
    NW = NC * NS
    nn = jnp.asarray(n_neighbors, i32).reshape(1, 1)
    rs = jnp.asarray(res_scale, f32).reshape(1, 1)
    tw2 = time_w.astype(f32).reshape(1, D)
    tb2 = time_b.astype(f32).reshape(1, D)
    R = 256

    outs = []
    for grp in (source_nodes, destination_nodes, negative_nodes):
        nodes = grp.astype(i32)
        Bs = nodes.shape[0]
        Ms = Bs * K
        g = _sc_gather_tables(nodes, tbl)
        nbrs3d = g[:, :K].reshape(NW, Ms // (NW * 128), 128)
        eidx2d = g[:, K:2 * K].reshape(Ms // 128, 128)
        nts = lax.bitcast_convert_type(g[:, 2 * K:3 * K], f32)
        x, nf_i = _sc_gather_embeddings(nodes, nbrs3d, node_emb, emb_bf3, Bs, Ms)
        nf = lax.bitcast_convert_type(nf_i, bf16).reshape(Ms, D)
        ef = _sc_gather_edges(eidx2d, edge_feat, Ms)
        emb = _tc_embed(R, Bs)(
            nn, rs, ts.reshape(Bs, 1), nts, x, nf, ef, tw2, tb2,
            Wq[:D], Wq[D:], Wk[:D].astype(bf16), Wk[D:D + DE], Wk[D + DE:],
            Wv[:D].astype(bf16), Wv[D:D + DE], Wv[D + DE:],
            W1[:D], W1[D:], b1.reshape(1, D), W2, b2.reshape(1, D))
        outs.append(emb)
    return tuple(outs)


# speedup vs baseline: 1.7003x; 1.7003x over previous
import functools
import math

import jax
import jax.numpy as jnp
from jax import lax
from jax.experimental import pallas as pl
from jax.experimental.pallas import tpu as pltpu
from jax.experimental.pallas import tpu_sc as plsc

D = 256
DE = 16
K = 20
H = 2
DH = D // H
TBLW = 64

_COS_C1 = 6.28125
_COS_C2 = 0.0019340515136718750
_COS_C3 = 1.2554227678489685e-06
_INV_2PI = 0.15915494309189535
_COS_POLY = (0.9999994, -0.49999544, 0.041660894, -0.001386227,
             2.424664e-05, -2.2163067e-07)


def _fast_cos(t):
    f = jnp.float32
    n = jnp.floor(t * f(_INV_2PI) + f(0.5))
    r = ((t - n * f(_COS_C1)) - n * f(_COS_C2)) - n * f(_COS_C3)
    z = r * r
    acc = jnp.full_like(z, f(_COS_POLY[-1]))
    for c in _COS_POLY[-2::-1]:
        acc = acc * z + f(c)
    return acc


def _sc_dims():
    try:
        info = plsc.get_sparse_core_info()
        return int(info.num_cores), int(info.num_subcores)
    except Exception:
        return 2, 16


def _sc_gather_tables(nodes, tbl):
    B3 = nodes.shape[0]
    NC, NS = _sc_dims()
    NW = NC * NS
    assert B3 % NW == 0
    per = B3 // NW
    mesh = plsc.VectorSubcoreMesh(
        core_axis_name="c", subcore_axis_name="s", num_cores=NC, num_subcores=NS)

    @functools.partial(
        pl.kernel,
        out_type=jax.ShapeDtypeStruct((B3, TBLW), jnp.int32),
        mesh=mesh,
        compiler_params=pltpu.CompilerParams(use_tc_tiling_on_sc=False),
        scratch_types=[
            pltpu.VMEM((per,), jnp.int32),
            pltpu.VMEM((per, TBLW), jnp.int32),
            pltpu.SemaphoreType.DMA,
        ],
    )
    def body(nodes_hbm, tbl_hbm, g_out, idx_v, tbl_v, s1):
        wid = lax.axis_index("s") * NC + lax.axis_index("c")
        base = wid * per
        pltpu.sync_copy(nodes_hbm.at[pl.ds(base, per)], idx_v)
        pltpu.async_copy(tbl_hbm.at[idx_v], tbl_v, s1).wait()
        pltpu.sync_copy(tbl_v, g_out.at[pl.ds(base, per)])

    return body(nodes, tbl)


def _sc_gather_embeddings(nodes, nidx3d, node_emb, emb_bf3, B3, M):
    NC, NS = _sc_dims()
    NW = NC * NS
    CH = 128
    assert B3 % NW == 0 and M % (NW * CH) == 0
    per_q = B3 // NW
    n_ch = M // (NW * CH)
    per = n_ch * CH
    mesh = plsc.VectorSubcoreMesh(
        core_axis_name="c", subcore_axis_name="s", num_cores=NC, num_subcores=NS)

    @functools.partial(
        pl.kernel,
        out_type=[
            jax.ShapeDtypeStruct((B3, D), jnp.float32),
            jax.ShapeDtypeStruct((M, 128), jnp.int32),
        ],
        mesh=mesh,
        scratch_types=[
            pltpu.VMEM((per_q,), jnp.int32),
            pltpu.VMEM((n_ch, CH), jnp.int32),
            pltpu.VMEM((per_q, D), jnp.float32),
            pltpu.VMEM((3, CH, 128), jnp.int32),
            pltpu.SemaphoreType.DMA,
            pltpu.SemaphoreType.DMA,
            pltpu.SemaphoreType.DMA,
            pltpu.SemaphoreType.DMA,
            pltpu.SemaphoreType.DMA,
            pltpu.SemaphoreType.DMA,
            pltpu.SemaphoreType.DMA,
        ],
    )
    def body(nodes_hbm, ni_hbm, emb_hbm, embbf_hbm, x_out, nf_out,
             qidx_v, ni_v, x_v, nbuf, sq, sg0, sg1, sg2, sw0, sw1, sw2):
        wid = lax.axis_index("s") * NC + lax.axis_index("c")
        qbase = wid * per_q
        base = wid * per
        pltpu.sync_copy(nodes_hbm.at[pl.ds(qbase, per_q)], qidx_v)
        pltpu.sync_copy(ni_hbm.at[wid], ni_v)
        cq = pltpu.async_copy(emb_hbm.at[qidx_v], x_v, sq)
        sg = (sg0, sg1, sg2)
        sw = (sw0, sw1, sw2)
        NB = 3
        gath = [None] * n_ch
        wb = [None] * n_ch
        for j in range(n_ch):
            b = j % NB
            if j >= NB:
                wb[j - NB].wait()
            gath[j] = pltpu.async_copy(embbf_hbm.at[ni_v.at[j]], nbuf.at[b], sg[b])
            if j >= 2:
                p = j - 2
                pb = p % NB
                gath[p].wait()
                wb[p] = pltpu.async_copy(
                    nbuf.at[pb], nf_out.at[pl.ds(base + p * CH, CH)], sw[pb])
        for p in (n_ch - 2, n_ch - 1):
            if p >= 0 and wb[p] is None:
                pb = p % NB
                gath[p].wait()
                wb[p] = pltpu.async_copy(
                    nbuf.at[pb], nf_out.at[pl.ds(base + p * CH, CH)], sw[pb])
        cq.wait()
        pltpu.sync_copy(x_v, x_out.at[pl.ds(qbase, per_q)])
        for p in range(max(0, n_ch - NB), n_ch):
            if wb[p] is not None:
                wb[p].wait()

    return body(nodes, nidx3d, node_emb, emb_bf3)


def _sc_gather_edges(eidx2d, edge_feat, M):
    NC, NS = _sc_dims()
    NW = NC * NS
    CH = 128
    assert M % (NW * CH) == 0
    n_ch = M // (NW * CH)
    per = n_ch * CH
    mesh = plsc.VectorSubcoreMesh(
        core_axis_name="c", subcore_axis_name="s", num_cores=NC, num_subcores=NS)

    @functools.partial(
        pl.kernel,
        out_type=jax.ShapeDtypeStruct((M, DE), jnp.float32),
        mesh=mesh,
        compiler_params=pltpu.CompilerParams(use_tc_tiling_on_sc=False),
        scratch_types=[
            pltpu.VMEM((n_ch, CH), jnp.int32),
            pltpu.VMEM((2, CH, DE), jnp.float32),
            pltpu.SemaphoreType.DMA,
            pltpu.SemaphoreType.DMA,
        ],
    )
    def body(ei_hbm, ef_hbm, ef_out, ei_v, ebuf, se0, se1):
        wid = lax.axis_index("s") * NC + lax.axis_index("c")
        base = wid * per
        pltpu.sync_copy(ei_hbm.at[pl.ds(wid * n_ch, n_ch)], ei_v)
        se = (se0, se1)
        prev = None
        for j in range(n_ch):
            b = j % 2
            ce = pltpu.async_copy(ef_hbm.at[ei_v.at[j]], ebuf.at[b], se[b])
            if prev is not None:
                pce, pj, pb = prev
                pce.wait()
                pltpu.sync_copy(ebuf.at[pb], ef_out.at[pl.ds(base + pj * CH, CH)])
            prev = (ce, j, b)
        pce, pj, pb = prev
        pce.wait()
        pltpu.sync_copy(ebuf.at[pb], ef_out.at[pl.ds(base + pj * CH, CH)])

    return body(eidx2d, edge_feat)


def _tc_body(nn_ref, rs_ref, ts_ref, nts_ref, x_ref, nf_ref, ef_ref,
             tw_ref, tb_ref, wqx_ref, wqt_ref, wkne_ref, wkno_ref, wke_ref,
             wkt_ref, wvne_ref, wvno_ref, wve_ref, wvt_ref, w1o_ref, w1x_ref,
             b1_ref, w2_ref, b2_ref, out_ref):
    R = ts_ref.shape[0]
    f32 = jnp.float32
    i32 = jnp.int32
    dot = functools.partial(jnp.dot, preferred_element_type=f32)
    x = x_ref[...]
    nf_i = nf_ref[...]
    nf_e = lax.bitcast_convert_type(nf_i << 16, f32)
    nf_o = lax.bitcast_convert_type(nf_i & i32(-65536), f32)
    ef = ef_ref[...]
    tw = tw_ref[...]
    tb = tb_ref[...]
    delta = ts_ref[...] - nts_ref[...]
    te = _fast_cos(delta[:, :, None] * tw.reshape(1, 1, D) + tb.reshape(1, 1, D))
    te = te.reshape(R * K, D)
    kk = (dot(nf_e, wkne_ref[...]) + dot(nf_o, wkno_ref[...])
          + dot(ef, wke_ref[...]) + dot(te, wkt_ref[...]))
    vv = (dot(nf_e, wvne_ref[...]) + dot(nf_o, wvno_ref[...])
          + dot(ef, wve_ref[...]) + dot(te, wvt_ref[...]))
    q = dot(x, wqx_ref[...]) + dot(jnp.cos(tb), wqt_ref[...])
    k3 = kk.reshape(R, K, D)
    v3 = vv.reshape(R, K, D)
    nn = nn_ref[0, 0]
    kiota = lax.broadcasted_iota(jnp.int32, (R, K), 1)
    inv_sqrt = f32(1.0 / math.sqrt(DH))
    outs = []
    for h in range(H):
        sl = slice(h * DH, (h + 1) * DH)
        q_h = q[:, sl]
        k_h = k3[:, :, sl]
        v_h = v3[:, :, sl]
        scores = jnp.sum(q_h[:, None, :] * k_h, axis=-1) * inv_sqrt
        scores = jnp.where(kiota < nn, scores, -jnp.inf)
        m = jnp.max(scores, axis=1, keepdims=True)
        e = jnp.exp(scores - m)
        attn = e / jnp.sum(e, axis=1, keepdims=True)
        outs.append(jnp.sum(attn[:, :, None] * v_h, axis=1))
    out = jnp.concatenate(outs, axis=-1)
    hh = dot(out, w1o_ref[...]) + dot(x, w1x_ref[...]) + b1_ref[...]
    hh = jnp.maximum(hh, 0.0)
    out_ref[...] = dot(hh, w2_ref[...]) + b2_ref[...] + rs_ref[0, 0] * x


def _tc_embed(R, B3, interpret=False):
    G = B3 // R
    row = lambda i: (i, 0)
    fix = lambda i: (0, 0)
    smem = pl.BlockSpec(memory_space=pltpu.SMEM)
    return pl.pallas_call(
        _tc_body,
        grid=(G,),
        in_specs=[
            smem,
            smem,
            pl.BlockSpec((R, 1), row),
            pl.BlockSpec((R, K), row),
            pl.BlockSpec((R, D), row),
            pl.BlockSpec((R * K, D // 2), row),
            pl.BlockSpec((R * K, DE), row),
            pl.BlockSpec((1, D), fix),
            pl.BlockSpec((1, D), fix),
            pl.BlockSpec((D, D), fix),
            pl.BlockSpec((D, D), fix),
            pl.BlockSpec((D // 2, D), fix),
            pl.BlockSpec((D // 2, D), fix),
            pl.BlockSpec((DE, D), fix),
            pl.BlockSpec((D, D), fix),
            pl.BlockSpec((D // 2, D), fix),
            pl.BlockSpec((D // 2, D), fix),
            pl.BlockSpec((DE, D), fix),
            pl.BlockSpec((D, D), fix),
            pl.BlockSpec((D, D), fix),
            pl.BlockSpec((D, D), fix),
            pl.BlockSpec((1, D), fix),
            pl.BlockSpec((D, D), fix),
            pl.BlockSpec((1, D), fix),
        ],
        out_specs=pl.BlockSpec((R, D), row),
        out_shape=jax.ShapeDtypeStruct((B3, D), jnp.float32),
        interpret=interpret,
    )


def kernel(source_nodes, destination_nodes, negative_nodes, edge_times, edge_idxs,
           n_neighbors, node_emb, edge_feat, nbr_nodes, nbr_eidx, nbr_times,
           time_w, time_b, Wq, Wk, Wv, W1, b1, W2, b2, res_scale):
    i32 = jnp.int32
    f32 = jnp.float32
    bf16 = jnp.bfloat16
    node_emb = node_emb.astype(f32)
    edge_feat = edge_feat.astype(f32)
    ts = edge_times.astype(f32)
    emb_bf3 = lax.bitcast_convert_type(
        node_emb.astype(bf16).reshape(node_emb.shape[0], 128, 2), i32)

    Nn = nbr_nodes.shape[0]
    tbl = jnp.concatenate([
        nbr_nodes.astype(i32),
        nbr_eidx.astype(i32),
        lax.bitcast_convert_type(nbr_times.astype(f32), i32),
        jnp.zeros((Nn, TBLW - 3 * K), i32),
    ], axis=1)

    NC, NS = _sc_dims()
    NW = NC * NS
    nn = jnp.asarray(n_neighbors, i32).reshape(1, 1)
    rs = jnp.asarray(res_scale, f32).reshape(1, 1)
    tw2 = time_w.astype(f32).reshape(1, D)
    tb2 = time_b.astype(f32).reshape(1, D)
    R = 256

    outs = []
    for grp in (source_nodes, destination_nodes, negative_nodes):
        nodes = grp.astype(i32)
        Bs = nodes.shape[0]
        Ms = Bs * K
        g = _sc_gather_tables(nodes, tbl)
        nbrs3d = g[:, :K].reshape(NW, Ms // (NW * 128), 128)
        eidx2d = g[:, K:2 * K].reshape(Ms // 128, 128)
        nts = lax.bitcast_convert_type(g[:, 2 * K:3 * K], f32)
        x, nf_i = _sc_gather_embeddings(nodes, nbrs3d, node_emb, emb_bf3, Bs, Ms)
        ef = _sc_gather_edges(eidx2d, edge_feat, Ms)
        emb = _tc_embed(R, Bs)(
            nn, rs, ts.reshape(Bs, 1), nts, x, nf_i, ef, tw2, tb2,
            Wq[:D], Wq[D:],
            Wk[0:D:2], Wk[1:D:2], Wk[D:D + DE], Wk[D + DE:],
            Wv[0:D:2], Wv[1:D:2], Wv[D:D + DE], Wv[D + DE:],
            W1[:D], W1[D:], b1.reshape(1, D), W2, b2.reshape(1, D))
        outs.append(emb)
    return tuple(outs)


# speedup vs baseline: 2.1656x; 1.2736x over previous
import functools
import math

import jax
import jax.numpy as jnp
from jax import lax
from jax.experimental import pallas as pl
from jax.experimental.pallas import tpu as pltpu
from jax.experimental.pallas import tpu_sc as plsc

D = 256
DE = 16
K = 20
H = 2
DH = D // H
TBLW = 64

_COS_C1 = 6.28125
_COS_C2 = 0.0019340515136718750
_COS_C3 = 1.2554227678489685e-06
_INV_2PI = 0.15915494309189535
_COS_POLY = (0.9999994, -0.49999544, 0.041660894, -0.001386227,
             2.424664e-05, -2.2163067e-07)


def _fast_cos(t):
    f = jnp.float32
    n = jnp.floor(t * f(_INV_2PI) + f(0.5))
    r = ((t - n * f(_COS_C1)) - n * f(_COS_C2)) - n * f(_COS_C3)
    z = r * r
    acc = jnp.full_like(z, f(_COS_POLY[-1]))
    for c in _COS_POLY[-2::-1]:
        acc = acc * z + f(c)
    return acc


def _sc_dims():
    try:
        info = plsc.get_sparse_core_info()
        return int(info.num_cores), int(info.num_subcores)
    except Exception:
        return 2, 16


def _sc_gather_tables(nodes, tbl):
    B3 = nodes.shape[0]
    NC, NS = _sc_dims()
    NW = NC * NS
    assert B3 % NW == 0
    per = B3 // NW
    mesh = plsc.VectorSubcoreMesh(
        core_axis_name="c", subcore_axis_name="s", num_cores=NC, num_subcores=NS)

    @functools.partial(
        pl.kernel,
        out_type=jax.ShapeDtypeStruct((B3, TBLW), jnp.int32),
        mesh=mesh,
        compiler_params=pltpu.CompilerParams(use_tc_tiling_on_sc=False),
        scratch_types=[
            pltpu.VMEM((per,), jnp.int32),
            pltpu.VMEM((per, TBLW), jnp.int32),
            pltpu.SemaphoreType.DMA,
        ],
    )
    def body(nodes_hbm, tbl_hbm, g_out, idx_v, tbl_v, s1):
        wid = lax.axis_index("s") * NC + lax.axis_index("c")
        base = wid * per
        pltpu.sync_copy(nodes_hbm.at[pl.ds(base, per)], idx_v)
        pltpu.async_copy(tbl_hbm.at[idx_v], tbl_v, s1).wait()
        pltpu.sync_copy(tbl_v, g_out.at[pl.ds(base, per)])

    return body(nodes, tbl)


def _sc_gather_embeddings(nodes, nidx3d, node_emb, emb_bf3, B3, M):
    NC, NS = _sc_dims()
    NW = NC * NS
    CH = 128
    assert B3 % NW == 0 and M % (NW * CH) == 0
    per_q = B3 // NW
    n_ch = M // (NW * CH)
    per = n_ch * CH
    mesh = plsc.VectorSubcoreMesh(
        core_axis_name="c", subcore_axis_name="s", num_cores=NC, num_subcores=NS)

    @functools.partial(
        pl.kernel,
        out_type=[
            jax.ShapeDtypeStruct((B3, D), jnp.float32),
            jax.ShapeDtypeStruct((M, 128), jnp.int32),
        ],
        mesh=mesh,
        scratch_types=[
            pltpu.VMEM((per_q,), jnp.int32),
            pltpu.VMEM((n_ch, CH), jnp.int32),
            pltpu.VMEM((per_q, D), jnp.float32),
            pltpu.VMEM((3, CH, 128), jnp.int32),
            pltpu.SemaphoreType.DMA,
            pltpu.SemaphoreType.DMA,
            pltpu.SemaphoreType.DMA,
            pltpu.SemaphoreType.DMA,
            pltpu.SemaphoreType.DMA,
            pltpu.SemaphoreType.DMA,
            pltpu.SemaphoreType.DMA,
        ],
    )
    def body(nodes_hbm, ni_hbm, emb_hbm, embbf_hbm, x_out, nf_out,
             qidx_v, ni_v, x_v, nbuf, sq, sg0, sg1, sg2, sw0, sw1, sw2):
        wid = lax.axis_index("s") * NC + lax.axis_index("c")
        qbase = wid * per_q
        base = wid * per
        pltpu.sync_copy(nodes_hbm.at[pl.ds(qbase, per_q)], qidx_v)
        pltpu.sync_copy(ni_hbm.at[wid], ni_v)
        cq = pltpu.async_copy(emb_hbm.at[qidx_v], x_v, sq)
        sg = (sg0, sg1, sg2)
        sw = (sw0, sw1, sw2)
        NB = 3
        gath = [None] * n_ch
        wb = [None] * n_ch
        for j in range(n_ch):
            b = j % NB
            if j >= NB:
                wb[j - NB].wait()
            gath[j] = pltpu.async_copy(embbf_hbm.at[ni_v.at[j]], nbuf.at[b], sg[b])
            if j >= 2:
                p = j - 2
                pb = p % NB
                gath[p].wait()
                wb[p] = pltpu.async_copy(
                    nbuf.at[pb], nf_out.at[pl.ds(base + p * CH, CH)], sw[pb])
        for p in (n_ch - 2, n_ch - 1):
            if p >= 0 and wb[p] is None:
                pb = p % NB
                gath[p].wait()
                wb[p] = pltpu.async_copy(
                    nbuf.at[pb], nf_out.at[pl.ds(base + p * CH, CH)], sw[pb])
        cq.wait()
        pltpu.sync_copy(x_v, x_out.at[pl.ds(qbase, per_q)])
        for p in range(max(0, n_ch - NB), n_ch):
            if wb[p] is not None:
                wb[p].wait()

    return body(nodes, nidx3d, node_emb, emb_bf3)


def _sc_gather_edges(eidx2d, edge_feat, M):
    NC, NS = _sc_dims()
    NW = NC * NS
    CH = 128
    assert M % (NW * CH) == 0
    n_ch = M // (NW * CH)
    per = n_ch * CH
    mesh = plsc.VectorSubcoreMesh(
        core_axis_name="c", subcore_axis_name="s", num_cores=NC, num_subcores=NS)

    @functools.partial(
        pl.kernel,
        out_type=jax.ShapeDtypeStruct((M, DE), jnp.float32),
        mesh=mesh,
        compiler_params=pltpu.CompilerParams(use_tc_tiling_on_sc=False),
        scratch_types=[
            pltpu.VMEM((n_ch, CH), jnp.int32),
            pltpu.VMEM((2, CH, DE), jnp.float32),
            pltpu.SemaphoreType.DMA,
            pltpu.SemaphoreType.DMA,
        ],
    )
    def body(ei_hbm, ef_hbm, ef_out, ei_v, ebuf, se0, se1):
        wid = lax.axis_index("s") * NC + lax.axis_index("c")
        base = wid * per
        pltpu.sync_copy(ei_hbm.at[pl.ds(wid * n_ch, n_ch)], ei_v)
        se = (se0, se1)
        prev = None
        for j in range(n_ch):
            b = j % 2
            ce = pltpu.async_copy(ef_hbm.at[ei_v.at[j]], ebuf.at[b], se[b])
            if prev is not None:
                pce, pj, pb = prev
                pce.wait()
                pltpu.sync_copy(ebuf.at[pb], ef_out.at[pl.ds(base + pj * CH, CH)])
            prev = (ce, j, b)
        pce, pj, pb = prev
        pce.wait()
        pltpu.sync_copy(ebuf.at[pb], ef_out.at[pl.ds(base + pj * CH, CH)])

    return body(eidx2d, edge_feat)


def _tc_body(nn_ref, rs_ref, ts_ref, nts_ref, x_ref, nf_ref, ef_ref,
             tw_ref, tb_ref, wqx_ref, wqt_ref, wkne_ref, wkno_ref, wke_ref,
             wkt_ref, wvne_ref, wvno_ref, wve_ref, wvt_ref, w1o_ref, w1x_ref,
             b1_ref, w2_ref, b2_ref, out_ref):
    R = ts_ref.shape[0]
    f32 = jnp.float32
    i32 = jnp.int32
    dot = functools.partial(jnp.dot, preferred_element_type=f32)
    x = x_ref[...]
    nf_i = nf_ref[...]
    nf_e = lax.bitcast_convert_type(nf_i << 16, f32)
    nf_o = lax.bitcast_convert_type(nf_i & i32(-65536), f32)
    ef = ef_ref[...]
    tw = tw_ref[...]
    tb = tb_ref[...]
    delta = ts_ref[...] - nts_ref[...]
    te = _fast_cos(delta[:, :, None] * tw.reshape(1, 1, D) + tb.reshape(1, 1, D))
    te = te.reshape(R * K, D)
    kk = (dot(nf_e, wkne_ref[...]) + dot(nf_o, wkno_ref[...])
          + dot(ef, wke_ref[...]) + dot(te, wkt_ref[...]))
    vv = (dot(nf_e, wvne_ref[...]) + dot(nf_o, wvno_ref[...])
          + dot(ef, wve_ref[...]) + dot(te, wvt_ref[...]))
    q = dot(x, wqx_ref[...]) + dot(jnp.cos(tb), wqt_ref[...])
    k3 = kk.reshape(R, K, D)
    v3 = vv.reshape(R, K, D)
    nn = nn_ref[0, 0]
    kiota = lax.broadcasted_iota(jnp.int32, (R, K), 1)
    inv_sqrt = f32(1.0 / math.sqrt(DH))
    outs = []
    for h in range(H):
        sl = slice(h * DH, (h + 1) * DH)
        q_h = q[:, sl]
        k_h = k3[:, :, sl]
        v_h = v3[:, :, sl]
        scores = jnp.sum(q_h[:, None, :] * k_h, axis=-1) * inv_sqrt
        scores = jnp.where(kiota < nn, scores, -jnp.inf)
        m = jnp.max(scores, axis=1, keepdims=True)
        e = jnp.exp(scores - m)
        attn = e / jnp.sum(e, axis=1, keepdims=True)
        outs.append(jnp.sum(attn[:, :, None] * v_h, axis=1))
    out = jnp.concatenate(outs, axis=-1)
    hh = dot(out, w1o_ref[...]) + dot(x, w1x_ref[...]) + b1_ref[...]
    hh = jnp.maximum(hh, 0.0)
    out_ref[...] = dot(hh, w2_ref[...]) + b2_ref[...] + rs_ref[0, 0] * x


def _tc_embed(R, B3, interpret=False):
    G = B3 // R
    row = lambda i: (i, 0)
    fix = lambda i: (0, 0)
    smem = pl.BlockSpec(memory_space=pltpu.SMEM)
    return pl.pallas_call(
        _tc_body,
        grid=(G,),
        in_specs=[
            smem,
            smem,
            pl.BlockSpec((R, 1), row),
            pl.BlockSpec((R, K), row),
            pl.BlockSpec((R, D), row),
            pl.BlockSpec((R * K, D // 2), row),
            pl.BlockSpec((R * K, DE), row),
            pl.BlockSpec((1, D), fix),
            pl.BlockSpec((1, D), fix),
            pl.BlockSpec((D, D), fix),
            pl.BlockSpec((D, D), fix),
            pl.BlockSpec((D // 2, D), fix),
            pl.BlockSpec((D // 2, D), fix),
            pl.BlockSpec((DE, D), fix),
            pl.BlockSpec((D, D), fix),
            pl.BlockSpec((D // 2, D), fix),
            pl.BlockSpec((D // 2, D), fix),
            pl.BlockSpec((DE, D), fix),
            pl.BlockSpec((D, D), fix),
            pl.BlockSpec((D, D), fix),
            pl.BlockSpec((D, D), fix),
            pl.BlockSpec((1, D), fix),
            pl.BlockSpec((D, D), fix),
            pl.BlockSpec((1, D), fix),
        ],
        out_specs=pl.BlockSpec((R, D), row),
        out_shape=jax.ShapeDtypeStruct((B3, D), jnp.float32),
        interpret=interpret,
    )


def kernel(source_nodes, destination_nodes, negative_nodes, edge_times, edge_idxs,
           n_neighbors, node_emb, edge_feat, nbr_nodes, nbr_eidx, nbr_times,
           time_w, time_b, Wq, Wk, Wv, W1, b1, W2, b2, res_scale):
    i32 = jnp.int32
    f32 = jnp.float32
    bf16 = jnp.bfloat16
    node_emb = node_emb.astype(f32)
    edge_feat = edge_feat.astype(f32)
    ts = edge_times.astype(f32)
    u = lax.bitcast_convert_type(node_emb, jnp.uint32)
    rbf = (u + jnp.uint32(0x7FFF) + ((u >> 16) & jnp.uint32(1))) >> 16
    emb_bf3 = lax.bitcast_convert_type(
        rbf[:, :128] | (rbf[:, 128:] << 16), i32)

    Nn = nbr_nodes.shape[0]
    tbl = jnp.concatenate([
        nbr_nodes.astype(i32),
        nbr_eidx.astype(i32),
        lax.bitcast_convert_type(nbr_times.astype(f32), i32),
        jnp.zeros((Nn, TBLW - 3 * K), i32),
    ], axis=1)

    NC, NS = _sc_dims()
    NW = NC * NS
    nn = jnp.asarray(n_neighbors, i32).reshape(1, 1)
    rs = jnp.asarray(res_scale, f32).reshape(1, 1)
    tw2 = time_w.astype(f32).reshape(1, D)
    tb2 = time_b.astype(f32).reshape(1, D)
    R = 256

    outs = []
    for grp in (source_nodes, destination_nodes, negative_nodes):
        nodes = grp.astype(i32)
        Bs = nodes.shape[0]
        Ms = Bs * K
        g = _sc_gather_tables(nodes, tbl)
        nbrs3d = g[:, :K].reshape(NW, Ms // (NW * 128), 128)
        eidx2d = g[:, K:2 * K].reshape(Ms // 128, 128)
        nts = lax.bitcast_convert_type(g[:, 2 * K:3 * K], f32)
        x, nf_i = _sc_gather_embeddings(nodes, nbrs3d, node_emb, emb_bf3, Bs, Ms)
        ef = _sc_gather_edges(eidx2d, edge_feat, Ms)
        emb = _tc_embed(R, Bs)(
            nn, rs, ts.reshape(Bs, 1), nts, x, nf_i, ef, tw2, tb2,
            Wq[:D], Wq[D:],
            Wk[:D // 2], Wk[D // 2:D], Wk[D:D + DE], Wk[D + DE:],
            Wv[:D // 2], Wv[D // 2:D], Wv[D:D + DE], Wv[D + DE:],
            W1[:D], W1[D:], b1.reshape(1, D), W2, b2.reshape(1, D))
        outs.append(emb)
    return tuple(outs)


# speedup vs baseline: 2.3662x; 1.0926x over previous
import functools
import math

import jax
import jax.numpy as jnp
from jax import lax
from jax.experimental import pallas as pl
from jax.experimental.pallas import tpu as pltpu
from jax.experimental.pallas import tpu_sc as plsc

D = 256
DE = 16
K = 20
H = 2
DH = D // H
TBLW = 64

_COS_C1 = 6.28125
_COS_C2 = 0.0019340515136718750
_COS_C3 = 1.2554227678489685e-06
_INV_2PI = 0.15915494309189535
_COS_POLY = (0.9999994, -0.49999544, 0.041660894, -0.001386227,
             2.424664e-05, -2.2163067e-07)


def _fast_cos(t):
    f = jnp.float32
    n = jnp.floor(t * f(_INV_2PI) + f(0.5))
    r = ((t - n * f(_COS_C1)) - n * f(_COS_C2)) - n * f(_COS_C3)
    z = r * r
    acc = jnp.full_like(z, f(_COS_POLY[-1]))
    for c in _COS_POLY[-2::-1]:
        acc = acc * z + f(c)
    return acc


def _sc_dims():
    try:
        info = plsc.get_sparse_core_info()
        return int(info.num_cores), int(info.num_subcores)
    except Exception:
        return 2, 16


def _sc_gather_tables(nodes, tbl):
    B3 = nodes.shape[0]
    NC, NS = _sc_dims()
    NW = NC * NS
    assert B3 % NW == 0
    per = B3 // NW
    mesh = plsc.VectorSubcoreMesh(
        core_axis_name="c", subcore_axis_name="s", num_cores=NC, num_subcores=NS)

    @functools.partial(
        pl.kernel,
        out_type=jax.ShapeDtypeStruct((B3, TBLW), jnp.int32),
        mesh=mesh,
        compiler_params=pltpu.CompilerParams(use_tc_tiling_on_sc=False),
        scratch_types=[
            pltpu.VMEM((per,), jnp.int32),
            pltpu.VMEM((per, TBLW), jnp.int32),
            pltpu.SemaphoreType.DMA,
        ],
    )
    def body(nodes_hbm, tbl_hbm, g_out, idx_v, tbl_v, s1):
        wid = lax.axis_index("s") * NC + lax.axis_index("c")
        base = wid * per
        pltpu.sync_copy(nodes_hbm.at[pl.ds(base, per)], idx_v)
        pltpu.async_copy(tbl_hbm.at[idx_v], tbl_v, s1).wait()
        pltpu.sync_copy(tbl_v, g_out.at[pl.ds(base, per)])

    return body(nodes, tbl)


def _sc_gather_embeddings(nodes, nidx3d, node_emb, B3, M):
    NC, NS = _sc_dims()
    NW = NC * NS
    CH = 128
    assert B3 % NW == 0 and M % (NW * CH) == 0
    per_q = B3 // NW
    n_ch = M // (NW * CH)
    per = n_ch * CH
    mesh = plsc.VectorSubcoreMesh(
        core_axis_name="c", subcore_axis_name="s", num_cores=NC, num_subcores=NS)

    @functools.partial(
        pl.kernel,
        out_type=[
            jax.ShapeDtypeStruct((B3, D), jnp.float32),
            jax.ShapeDtypeStruct((M, D), jnp.float32),
        ],
        mesh=mesh,
        scratch_types=[
            pltpu.VMEM((per_q,), jnp.int32),
            pltpu.VMEM((n_ch, CH), jnp.int32),
            pltpu.VMEM((per_q, D), jnp.float32),
            pltpu.VMEM((3, CH, D), jnp.float32),
            pltpu.SemaphoreType.DMA,
            pltpu.SemaphoreType.DMA,
            pltpu.SemaphoreType.DMA,
            pltpu.SemaphoreType.DMA,
            pltpu.SemaphoreType.DMA,
            pltpu.SemaphoreType.DMA,
            pltpu.SemaphoreType.DMA,
        ],
    )
    def body(nodes_hbm, ni_hbm, emb_hbm, x_out, nf_out,
             qidx_v, ni_v, x_v, nbuf, sq, sg0, sg1, sg2, sw0, sw1, sw2):
        wid = lax.axis_index("s") * NC + lax.axis_index("c")
        qbase = wid * per_q
        base = wid * per
        pltpu.sync_copy(nodes_hbm.at[pl.ds(qbase, per_q)], qidx_v)
        pltpu.sync_copy(ni_hbm.at[wid], ni_v)
        cq = pltpu.async_copy(emb_hbm.at[qidx_v], x_v, sq)
        sg = (sg0, sg1, sg2)
        sw = (sw0, sw1, sw2)
        NB = 3
        gath = [None] * n_ch
        wb = [None] * n_ch
        for j in range(n_ch):
            b = j % NB
            if j >= NB:
                wb[j - NB].wait()
            gath[j] = pltpu.async_copy(emb_hbm.at[ni_v.at[j]], nbuf.at[b], sg[b])
            if j >= 2:
                p = j - 2
                pb = p % NB
                gath[p].wait()
                wb[p] = pltpu.async_copy(
                    nbuf.at[pb], nf_out.at[pl.ds(base + p * CH, CH)], sw[pb])
        for p in (n_ch - 2, n_ch - 1):
            if p >= 0 and wb[p] is None:
                pb = p % NB
                gath[p].wait()
                wb[p] = pltpu.async_copy(
                    nbuf.at[pb], nf_out.at[pl.ds(base + p * CH, CH)], sw[pb])
        cq.wait()
        pltpu.sync_copy(x_v, x_out.at[pl.ds(qbase, per_q)])
        for p in range(max(0, n_ch - NB), n_ch):
            if wb[p] is not None:
                wb[p].wait()

    return body(nodes, nidx3d, node_emb)


def _sc_gather_edges(eidx2d, edge_feat, M):
    NC, NS = _sc_dims()
    NW = NC * NS
    CH = 128
    assert M % (NW * CH) == 0
    n_ch = M // (NW * CH)
    per = n_ch * CH
    mesh = plsc.VectorSubcoreMesh(
        core_axis_name="c", subcore_axis_name="s", num_cores=NC, num_subcores=NS)

    @functools.partial(
        pl.kernel,
        out_type=jax.ShapeDtypeStruct((M, DE), jnp.float32),
        mesh=mesh,
        compiler_params=pltpu.CompilerParams(use_tc_tiling_on_sc=False),
        scratch_types=[
            pltpu.VMEM((n_ch, CH), jnp.int32),
            pltpu.VMEM((2, CH, DE), jnp.float32),
            pltpu.SemaphoreType.DMA,
            pltpu.SemaphoreType.DMA,
        ],
    )
    def body(ei_hbm, ef_hbm, ef_out, ei_v, ebuf, se0, se1):
        wid = lax.axis_index("s") * NC + lax.axis_index("c")
        base = wid * per
        pltpu.sync_copy(ei_hbm.at[pl.ds(wid * n_ch, n_ch)], ei_v)
        se = (se0, se1)
        prev = None
        for j in range(n_ch):
            b = j % 2
            ce = pltpu.async_copy(ef_hbm.at[ei_v.at[j]], ebuf.at[b], se[b])
            if prev is not None:
                pce, pj, pb = prev
                pce.wait()
                pltpu.sync_copy(ebuf.at[pb], ef_out.at[pl.ds(base + pj * CH, CH)])
            prev = (ce, j, b)
        pce, pj, pb = prev
        pce.wait()
        pltpu.sync_copy(ebuf.at[pb], ef_out.at[pl.ds(base + pj * CH, CH)])

    return body(eidx2d, edge_feat)


def _tc_body(nn_ref, rs_ref, ts_ref, nts_ref, x_ref, nf_ref, ef_ref,
             tw_ref, tb_ref, wqx_ref, wqt_ref, wkn_ref, wke_ref,
             wkt_ref, wvn_ref, wve_ref, wvt_ref, w1o_ref, w1x_ref,
             b1_ref, w2_ref, b2_ref, out_ref):
    R = ts_ref.shape[0]
    f32 = jnp.float32
    dot = functools.partial(jnp.dot, preferred_element_type=f32)
    x = x_ref[...]
    nf = nf_ref[...]
    ef = ef_ref[...]
    tw = tw_ref[...]
    tb = tb_ref[...]
    delta = ts_ref[...] - nts_ref[...]
    te = _fast_cos(delta[:, :, None] * tw.reshape(1, 1, D) + tb.reshape(1, 1, D))
    te = te.reshape(R * K, D)
    kk = dot(nf, wkn_ref[...]) + dot(ef, wke_ref[...]) + dot(te, wkt_ref[...])
    vv = dot(nf, wvn_ref[...]) + dot(ef, wve_ref[...]) + dot(te, wvt_ref[...])
    q = dot(x, wqx_ref[...]) + dot(jnp.cos(tb), wqt_ref[...])
    k3 = kk.reshape(R, K, D)
    v3 = vv.reshape(R, K, D)
    nn = nn_ref[0, 0]
    kiota = lax.broadcasted_iota(jnp.int32, (R, K), 1)
    inv_sqrt = f32(1.0 / math.sqrt(DH))
    outs = []
    for h in range(H):
        sl = slice(h * DH, (h + 1) * DH)
        q_h = q[:, sl]
        k_h = k3[:, :, sl]
        v_h = v3[:, :, sl]
        scores = jnp.sum(q_h[:, None, :] * k_h, axis=-1) * inv_sqrt
        scores = jnp.where(kiota < nn, scores, -jnp.inf)
        m = jnp.max(scores, axis=1, keepdims=True)
        e = jnp.exp(scores - m)
        attn = e / jnp.sum(e, axis=1, keepdims=True)
        outs.append(jnp.sum(attn[:, :, None] * v_h, axis=1))
    out = jnp.concatenate(outs, axis=-1)
    hh = dot(out, w1o_ref[...]) + dot(x, w1x_ref[...]) + b1_ref[...]
    hh = jnp.maximum(hh, 0.0)
    out_ref[...] = dot(hh, w2_ref[...]) + b2_ref[...] + rs_ref[0, 0] * x


def _tc_embed(R, B3, interpret=False):
    G = B3 // R
    row = lambda i: (i, 0)
    fix = lambda i: (0, 0)
    smem = pl.BlockSpec(memory_space=pltpu.SMEM)
    return pl.pallas_call(
        _tc_body,
        grid=(G,),
        in_specs=[
            smem,
            smem,
            pl.BlockSpec((R, 1), row),
            pl.BlockSpec((R, K), row),
            pl.BlockSpec((R, D), row),
            pl.BlockSpec((R * K, D), row),
            pl.BlockSpec((R * K, DE), row),
            pl.BlockSpec((1, D), fix),
            pl.BlockSpec((1, D), fix),
            pl.BlockSpec((D, D), fix),
            pl.BlockSpec((D, D), fix),
            pl.BlockSpec((D, D), fix),
            pl.BlockSpec((DE, D), fix),
            pl.BlockSpec((D, D), fix),
            pl.BlockSpec((D, D), fix),
            pl.BlockSpec((DE, D), fix),
            pl.BlockSpec((D, D), fix),
            pl.BlockSpec((D, D), fix),
            pl.BlockSpec((D, D), fix),
            pl.BlockSpec((1, D), fix),
            pl.BlockSpec((D, D), fix),
            pl.BlockSpec((1, D), fix),
        ],
        out_specs=pl.BlockSpec((R, D), row),
        out_shape=jax.ShapeDtypeStruct((B3, D), jnp.float32),
        interpret=interpret,
    )


def kernel(source_nodes, destination_nodes, negative_nodes, edge_times, edge_idxs,
           n_neighbors, node_emb, edge_feat, nbr_nodes, nbr_eidx, nbr_times,
           time_w, time_b, Wq, Wk, Wv, W1, b1, W2, b2, res_scale):
    i32 = jnp.int32
    f32 = jnp.float32
    bf16 = jnp.bfloat16
    node_emb = node_emb.astype(f32)
    edge_feat = edge_feat.astype(f32)
    ts = edge_times.astype(f32)

    Nn = nbr_nodes.shape[0]
    tbl = jnp.concatenate([
        nbr_nodes.astype(i32),
        nbr_eidx.astype(i32),
        lax.bitcast_convert_type(nbr_times.astype(f32), i32),
        jnp.zeros((Nn, TBLW - 3 * K), i32),
    ], axis=1)

    NC, NS = _sc_dims()
    NW = NC * NS
    nn = jnp.asarray(n_neighbors, i32).reshape(1, 1)
    rs = jnp.asarray(res_scale, f32).reshape(1, 1)
    tw2 = time_w.astype(f32).reshape(1, D)
    tb2 = time_b.astype(f32).reshape(1, D)
    R = 256

    outs = []
    for grp in (source_nodes, destination_nodes, negative_nodes):
        nodes = grp.astype(i32)
        Bs = nodes.shape[0]
        Ms = Bs * K
        g = _sc_gather_tables(nodes, tbl)
        nbrs3d = g[:, :K].reshape(NW, Ms // (NW * 128), 128)
        eidx2d = g[:, K:2 * K].reshape(Ms // 128, 128)
        nts = lax.bitcast_convert_type(g[:, 2 * K:3 * K], f32)
        x, nf = _sc_gather_embeddings(nodes, nbrs3d, node_emb, Bs, Ms)
        ef = _sc_gather_edges(eidx2d, edge_feat, Ms)
        emb = _tc_embed(R, Bs)(
            nn, rs, ts.reshape(Bs, 1), nts, x, nf, ef, tw2, tb2,
            Wq[:D], Wq[D:], Wk[:D], Wk[D:D + DE], Wk[D + DE:],
            Wv[:D], Wv[D:D + DE], Wv[D + DE:],
            W1[:D], W1[D:], b1.reshape(1, D), W2, b2.reshape(1, D))
        outs.append(emb)
    return tuple(outs)


# speedup vs baseline: 2.3793x; 1.0055x over previous
import functools
import math

import jax
import jax.numpy as jnp
from jax import lax
from jax.experimental import pallas as pl
from jax.experimental.pallas import tpu as pltpu
from jax.experimental.pallas import tpu_sc as plsc

D = 256
DE = 16
K = 20
H = 2
DH = D // H
TBLW = 64

_COS_C1 = 6.28125
_COS_C2 = 0.0019340515136718750
_COS_C3 = 1.2554227678489685e-06
_INV_2PI = 0.15915494309189535
_COS_POLY = (0.9999994, -0.49999544, 0.041660894, -0.001386227,
             2.424664e-05, -2.2163067e-07)


def _fast_cos(t):
    f = jnp.float32
    n = jnp.floor(t * f(_INV_2PI) + f(0.5))
    r = ((t - n * f(_COS_C1)) - n * f(_COS_C2)) - n * f(_COS_C3)
    z = r * r
    acc = jnp.full_like(z, f(_COS_POLY[-1]))
    for c in _COS_POLY[-2::-1]:
        acc = acc * z + f(c)
    return acc


def _sc_dims():
    try:
        info = plsc.get_sparse_core_info()
        return int(info.num_cores), int(info.num_subcores)
    except Exception:
        return 2, 16


def _sc_gather_tables(nodes, tbl):
    B3 = nodes.shape[0]
    NC, NS = _sc_dims()
    NW = NC * NS
    assert B3 % NW == 0
    per = B3 // NW
    mesh = plsc.VectorSubcoreMesh(
        core_axis_name="c", subcore_axis_name="s", num_cores=NC, num_subcores=NS)

    @functools.partial(
        pl.kernel,
        out_type=jax.ShapeDtypeStruct((B3, TBLW), jnp.int32),
        mesh=mesh,
        compiler_params=pltpu.CompilerParams(use_tc_tiling_on_sc=False),
        scratch_types=[
            pltpu.VMEM((per,), jnp.int32),
            pltpu.VMEM((per, TBLW), jnp.int32),
            pltpu.SemaphoreType.DMA,
        ],
    )
    def body(nodes_hbm, tbl_hbm, g_out, idx_v, tbl_v, s1):
        wid = lax.axis_index("s") * NC + lax.axis_index("c")
        base = wid * per
        pltpu.sync_copy(nodes_hbm.at[pl.ds(base, per)], idx_v)
        pltpu.async_copy(tbl_hbm.at[idx_v], tbl_v, s1).wait()
        pltpu.sync_copy(tbl_v, g_out.at[pl.ds(base, per)])

    return body(nodes, tbl)


def _sc_gather_embeddings(nodes, nidx3d, node_emb, B3, M):
    NC, NS = _sc_dims()
    NW = NC * NS
    CH = 128
    assert B3 % NW == 0 and M % (NW * CH) == 0
    per_q = B3 // NW
    n_ch = M // (NW * CH)
    per = n_ch * CH
    mesh = plsc.VectorSubcoreMesh(
        core_axis_name="c", subcore_axis_name="s", num_cores=NC, num_subcores=NS)

    @functools.partial(
        pl.kernel,
        out_type=[
            jax.ShapeDtypeStruct((B3, D), jnp.float32),
            jax.ShapeDtypeStruct((M, D), jnp.float32),
        ],
        mesh=mesh,
        scratch_types=[
            pltpu.VMEM((per_q,), jnp.int32),
            pltpu.VMEM((n_ch, CH), jnp.int32),
            pltpu.VMEM((per_q, D), jnp.float32),
            pltpu.VMEM((3, CH, D), jnp.float32),
            pltpu.SemaphoreType.DMA,
            pltpu.SemaphoreType.DMA,
            pltpu.SemaphoreType.DMA,
            pltpu.SemaphoreType.DMA,
            pltpu.SemaphoreType.DMA,
            pltpu.SemaphoreType.DMA,
            pltpu.SemaphoreType.DMA,
        ],
    )
    def body(nodes_hbm, ni_hbm, emb_hbm, x_out, nf_out,
             qidx_v, ni_v, x_v, nbuf, sq, sg0, sg1, sg2, sw0, sw1, sw2):
        wid = lax.axis_index("s") * NC + lax.axis_index("c")
        qbase = wid * per_q
        base = wid * per
        pltpu.sync_copy(nodes_hbm.at[pl.ds(qbase, per_q)], qidx_v)
        pltpu.sync_copy(ni_hbm.at[wid], ni_v)
        cq = pltpu.async_copy(emb_hbm.at[qidx_v], x_v, sq)
        sg = (sg0, sg1, sg2)
        sw = (sw0, sw1, sw2)
        NB = 3
        gath = [None] * n_ch
        wb = [None] * n_ch
        for j in range(n_ch):
            b = j % NB
            if j >= NB:
                wb[j - NB].wait()
            gath[j] = pltpu.async_copy(emb_hbm.at[ni_v.at[j]], nbuf.at[b], sg[b])
            if j >= 2:
                p = j - 2
                pb = p % NB
                gath[p].wait()
                wb[p] = pltpu.async_copy(
                    nbuf.at[pb], nf_out.at[pl.ds(base + p * CH, CH)], sw[pb])
        for p in (n_ch - 2, n_ch - 1):
            if p >= 0 and wb[p] is None:
                pb = p % NB
                gath[p].wait()
                wb[p] = pltpu.async_copy(
                    nbuf.at[pb], nf_out.at[pl.ds(base + p * CH, CH)], sw[pb])
        cq.wait()
        pltpu.sync_copy(x_v, x_out.at[pl.ds(qbase, per_q)])
        for p in range(max(0, n_ch - NB), n_ch):
            if wb[p] is not None:
                wb[p].wait()

    return body(nodes, nidx3d, node_emb)


def _sc_gather_edges(eidx2d, edge_feat, M):
    NC, NS = _sc_dims()
    NW = NC * NS
    CH = 128
    assert M % (NW * CH) == 0
    n_ch = M // (NW * CH)
    per = n_ch * CH
    mesh = plsc.VectorSubcoreMesh(
        core_axis_name="c", subcore_axis_name="s", num_cores=NC, num_subcores=NS)

    @functools.partial(
        pl.kernel,
        out_type=jax.ShapeDtypeStruct((M, DE), jnp.float32),
        mesh=mesh,
        compiler_params=pltpu.CompilerParams(use_tc_tiling_on_sc=False),
        scratch_types=[
            pltpu.VMEM((n_ch, CH), jnp.int32),
            pltpu.VMEM((2, CH, DE), jnp.float32),
            pltpu.SemaphoreType.DMA,
            pltpu.SemaphoreType.DMA,
        ],
    )
    def body(ei_hbm, ef_hbm, ef_out, ei_v, ebuf, se0, se1):
        wid = lax.axis_index("s") * NC + lax.axis_index("c")
        base = wid * per
        pltpu.sync_copy(ei_hbm.at[pl.ds(wid * n_ch, n_ch)], ei_v)
        se = (se0, se1)
        prev = None
        for j in range(n_ch):
            b = j % 2
            ce = pltpu.async_copy(ef_hbm.at[ei_v.at[j]], ebuf.at[b], se[b])
            if prev is not None:
                pce, pj, pb = prev
                pce.wait()
                pltpu.sync_copy(ebuf.at[pb], ef_out.at[pl.ds(base + pj * CH, CH)])
            prev = (ce, j, b)
        pce, pj, pb = prev
        pce.wait()
        pltpu.sync_copy(ebuf.at[pb], ef_out.at[pl.ds(base + pj * CH, CH)])

    return body(eidx2d, edge_feat)


def _tc_body(nn_ref, rs_ref, ts_ref, nts_ref, x_ref, nf_ref, ef_ref,
             tw_ref, tb_ref, wqx_ref, wqt_ref, wkn_ref, wke_ref,
             wkt_ref, wvn_ref, wve_ref, wvt_ref, w1o_ref, w1x_ref,
             b1_ref, w2_ref, b2_ref, out_ref):
    R = ts_ref.shape[0]
    f32 = jnp.float32
    dot = functools.partial(jnp.dot, preferred_element_type=f32)
    x = x_ref[...]
    nf = nf_ref[...]
    ef = ef_ref[...]
    tw = tw_ref[...]
    tb = tb_ref[...]
    delta = ts_ref[...] - nts_ref[...]
    te = _fast_cos(delta[:, :, None] * tw.reshape(1, 1, D) + tb.reshape(1, 1, D))
    te = te.reshape(R * K, D)
    kk = dot(nf, wkn_ref[...]) + dot(ef, wke_ref[...]) + dot(te, wkt_ref[...])
    vv = dot(nf, wvn_ref[...]) + dot(ef, wve_ref[...]) + dot(te, wvt_ref[...])
    q = dot(x, wqx_ref[...]) + dot(jnp.cos(tb), wqt_ref[...])
    k3 = kk.reshape(R, K, D)
    v3 = vv.reshape(R, K, D)
    nn = nn_ref[0, 0]
    kiota = lax.broadcasted_iota(jnp.int32, (R, K), 1)
    inv_sqrt = f32(1.0 / math.sqrt(DH))
    outs = []
    for h in range(H):
        sl = slice(h * DH, (h + 1) * DH)
        q_h = q[:, sl]
        k_h = k3[:, :, sl]
        v_h = v3[:, :, sl]
        scores = jnp.sum(q_h[:, None, :] * k_h, axis=-1) * inv_sqrt
        scores = jnp.where(kiota < nn, scores, -jnp.inf)
        m = jnp.max(scores, axis=1, keepdims=True)
        e = jnp.exp(scores - m)
        attn = e / jnp.sum(e, axis=1, keepdims=True)
        outs.append(jnp.sum(attn[:, :, None] * v_h, axis=1))
    out = jnp.concatenate(outs, axis=-1)
    hh = dot(out, w1o_ref[...]) + dot(x, w1x_ref[...]) + b1_ref[...]
    hh = jnp.maximum(hh, 0.0)
    out_ref[...] = dot(hh, w2_ref[...]) + b2_ref[...] + rs_ref[0, 0] * x


def _tc_embed(R, B3, interpret=False):
    G = B3 // R
    row = lambda i: (i, 0)
    fix = lambda i: (0, 0)
    smem = pl.BlockSpec(memory_space=pltpu.SMEM)
    return pl.pallas_call(
        _tc_body,
        grid=(G,),
        in_specs=[
            smem,
            smem,
            pl.BlockSpec((R, 1), row),
            pl.BlockSpec((R, K), row),
            pl.BlockSpec((R, D), row),
            pl.BlockSpec((R * K, D), row),
            pl.BlockSpec((R * K, DE), row),
            pl.BlockSpec((1, D), fix),
            pl.BlockSpec((1, D), fix),
            pl.BlockSpec((D, D), fix),
            pl.BlockSpec((D, D), fix),
            pl.BlockSpec((D, D), fix),
            pl.BlockSpec((DE, D), fix),
            pl.BlockSpec((D, D), fix),
            pl.BlockSpec((D, D), fix),
            pl.BlockSpec((DE, D), fix),
            pl.BlockSpec((D, D), fix),
            pl.BlockSpec((D, D), fix),
            pl.BlockSpec((D, D), fix),
            pl.BlockSpec((1, D), fix),
            pl.BlockSpec((D, D), fix),
            pl.BlockSpec((1, D), fix),
        ],
        out_specs=pl.BlockSpec((R, D), row),
        out_shape=jax.ShapeDtypeStruct((B3, D), jnp.float32),
        interpret=interpret,
    )


def kernel(source_nodes, destination_nodes, negative_nodes, edge_times, edge_idxs,
           n_neighbors, node_emb, edge_feat, nbr_nodes, nbr_eidx, nbr_times,
           time_w, time_b, Wq, Wk, Wv, W1, b1, W2, b2, res_scale):
    i32 = jnp.int32
    f32 = jnp.float32
    bf16 = jnp.bfloat16
    node_emb = node_emb.astype(f32)
    edge_feat = edge_feat.astype(f32)
    ts = edge_times.astype(f32)

    Nn = nbr_nodes.shape[0]
    tbl = jnp.concatenate([
        nbr_nodes.astype(i32),
        nbr_eidx.astype(i32),
        lax.bitcast_convert_type(nbr_times.astype(f32), i32),
        jnp.zeros((Nn, TBLW - 3 * K), i32),
    ], axis=1)

    NC, NS = _sc_dims()
    NW = NC * NS
    nn = jnp.asarray(n_neighbors, i32).reshape(1, 1)
    rs = jnp.asarray(res_scale, f32).reshape(1, 1)
    tw2 = time_w.astype(f32).reshape(1, D)
    tb2 = time_b.astype(f32).reshape(1, D)
    R = 256

    groups = [source_nodes.astype(i32), destination_nodes.astype(i32),
              negative_nodes.astype(i32)]
    Bs = groups[0].shape[0]
    Ms = Bs * K
    allnodes = jnp.concatenate(groups)
    g_all = _sc_gather_tables(allnodes, tbl)
    stage = []
    for p, nodes in enumerate(groups):
        g = g_all[p * Bs:(p + 1) * Bs]
        nbrs3d = g[:, :K].reshape(NW, Ms // (NW * 128), 128)
        eidx2d = g[:, K:2 * K].reshape(Ms // 128, 128)
        nts = lax.bitcast_convert_type(g[:, 2 * K:3 * K], f32)
        x, nf = _sc_gather_embeddings(nodes, nbrs3d, node_emb, Bs, Ms)
        ef = _sc_gather_edges(eidx2d, edge_feat, Ms)
        stage.append((nts, x, nf, ef))
    outs = []
    for nts, x, nf, ef in stage:
        emb = _tc_embed(R, Bs)(
            nn, rs, ts.reshape(Bs, 1), nts, x, nf, ef, tw2, tb2,
            Wq[:D], Wq[D:], Wk[:D], Wk[D:D + DE], Wk[D + DE:],
            Wv[:D], Wv[D:D + DE], Wv[D + DE:],
            W1[:D], W1[D:], b1.reshape(1, D), W2, b2.reshape(1, D))
        outs.append(emb)
    return tuple(outs)
